# baseline (speedup 1.0000x reference)
import jax
import jax.numpy as jnp
from jax import lax
from jax.experimental import pallas as pl
from jax.experimental.pallas import tpu as pltpu
from jax.experimental.pallas import tpu_sc as plsc

B = 16384
H = 50
LEN_RULES = 100000
PAD_TOK = LEN_RULES
TBL_ALLOC = 100016
NC, NS, L = 2, 16, 16
NW = NC * NS
ROWS_PER_W = B // NW
GROUPS = ROWS_PER_W // L
NACC = 4


def _body(rules_hbm, table_hbm, out_hbm, table_v, idx_v, out_v):
    wid = lax.axis_index("s") * NC + lax.axis_index("c")
    pltpu.sync_copy(table_hbm.at[pl.ds(0, LEN_RULES)],
                    table_v.at[pl.ds(0, LEN_RULES)])
    pltpu.sync_copy(rules_hbm.at[pl.ds(wid * ROWS_PER_W, ROWS_PER_W)], idx_v)

    lanes = lax.iota(jnp.int32, L)
    plsc.store_scatter(
        table_v,
        [jnp.full((L,), PAD_TOK, jnp.int32)],
        jnp.full((L,), -jnp.inf, jnp.float32),
        mask=lanes == 0,
    )

    def group(g, _):
        rows = g * L + lanes
        acc = [jnp.ones((L,), jnp.float32) for _ in range(NACC)]
        for j in range(H):
            iv = plsc.load_gather(idx_v, [rows, jnp.full((L,), j, jnp.int32)])
            v = plsc.load_gather(table_v, [iv])
            acc[j % NACC] = acc[j % NACC] * (1.0 + jnp.exp(v))
        p = (acc[0] * acc[1]) * (acc[2] * acc[3])
        no = 1.0 - 1.0 / p
        no = jnp.minimum(jnp.maximum(no, 0.0001), 0.99999)
        out_v[pl.ds(g * L, L)] = no
        return 0

    lax.fori_loop(0, GROUPS, group, 0)
    pltpu.sync_copy(out_v, out_hbm.at[pl.ds(wid * ROWS_PER_W, ROWS_PER_W)])


@jax.jit
def kernel(rules, relation, table):
    del relation
    run = pl.kernel(
        _body,
        out_type=jax.ShapeDtypeStruct((B,), jnp.float32),
        mesh=plsc.VectorSubcoreMesh(
            core_axis_name="c", subcore_axis_name="s",
            num_cores=NC, num_subcores=NS,
        ),
        compiler_params=pltpu.CompilerParams(
            needs_layout_passes=False, use_tc_tiling_on_sc=False),
        scratch_types=[
            pltpu.VMEM((TBL_ALLOC,), jnp.float32),
            pltpu.VMEM((ROWS_PER_W, H), jnp.int32),
            pltpu.VMEM((ROWS_PER_W,), jnp.float32),
        ],
    )
    return run(rules, table[:, 0]).reshape(B, 1)


# speedup vs baseline: 1.0034x; 1.0034x over previous
import jax
import jax.numpy as jnp
from jax import lax
from jax.experimental import pallas as pl
from jax.experimental.pallas import tpu as pltpu
from jax.experimental.pallas import tpu_sc as plsc

B = 16384
H = 50
LEN_RULES = 100000
PAD_TOK = LEN_RULES
TBL_PAD = 100008
LOG2E = 1.4426950408889634
NC, NS, L = 2, 16, 16
NW = NC * NS
ROWS_PER_W = B // NW
GROUPS = ROWS_PER_W // L
NACC = 4


def _body(rules_hbm, table_hbm, out_hbm, table_v, idx_v, out_v):
    wid = lax.axis_index("s") * NC + lax.axis_index("c")
    pltpu.sync_copy(table_hbm, table_v)
    pltpu.sync_copy(rules_hbm.at[pl.ds(wid * ROWS_PER_W, ROWS_PER_W)], idx_v)

    lanes = lax.iota(jnp.int32, L)

    def group(g, _):
        rows = g * L + lanes
        acc = [jnp.ones((L,), jnp.float32) for _ in range(NACC)]
        for j in range(H):
            iv = plsc.load_gather(idx_v, [rows, jnp.full((L,), j, jnp.int32)])
            v = plsc.load_gather(table_v, [iv])
            acc[j % NACC] = acc[j % NACC] * (1.0 + jnp.exp(v))
        p = (acc[0] * acc[1]) * (acc[2] * acc[3])
        no = 1.0 - 1.0 / p
        no = jnp.minimum(jnp.maximum(no, 0.0001), 0.99999)
        out_v[pl.ds(g * L, L)] = no
        return 0

    lax.fori_loop(0, GROUPS, group, 0)
    pltpu.sync_copy(out_v, out_hbm.at[pl.ds(wid * ROWS_PER_W, ROWS_PER_W)])


@jax.jit
def kernel(rules, relation, table):
    del relation
    tbl = table[:, 0].at[PAD_TOK].set(-jnp.inf)
    tbl = jnp.concatenate([tbl, jnp.zeros((TBL_PAD - (LEN_RULES + 1),), jnp.float32)])
    run = pl.kernel(
        _body,
        out_type=jax.ShapeDtypeStruct((B,), jnp.float32),
        mesh=plsc.VectorSubcoreMesh(
            core_axis_name="c", subcore_axis_name="s",
            num_cores=NC, num_subcores=NS,
        ),
        compiler_params=pltpu.CompilerParams(
            needs_layout_passes=False, use_tc_tiling_on_sc=False),
        scratch_types=[
            pltpu.VMEM((TBL_PAD,), jnp.float32),
            pltpu.VMEM((ROWS_PER_W, H), jnp.int32),
            pltpu.VMEM((ROWS_PER_W,), jnp.float32),
        ],
    )
    return run(rules, tbl).reshape(B, 1)


# speedup vs baseline: 1.0572x; 1.0537x over previous
import jax
import jax.numpy as jnp
from jax import lax
from jax.experimental import pallas as pl
from jax.experimental.pallas import tpu as pltpu
from jax.experimental.pallas import tpu_sc as plsc

B = 16384
H = 50
LEN_RULES = 100000
PAD_TOK = LEN_RULES
TBL_PAD = 100008
NC, NS, L = 2, 16, 16
NW = NC * NS
ROWS_PER_W = B // NW
CHUNK = 128
NCHUNK = ROWS_PER_W // CHUNK
GROUPS_PER_CHUNK = CHUNK // L
NACC = 4


def _body(rules_hbm, table_hbm, out_hbm, table_v, idx_v, out_v):
    wid = lax.axis_index("s") * NC + lax.axis_index("c")
    base = wid * ROWS_PER_W
    pltpu.sync_copy(table_hbm, table_v)

    lanes = lax.iota(jnp.int32, L)

    for c in range(NCHUNK):
        pltpu.sync_copy(rules_hbm.at[pl.ds(base + c * CHUNK, CHUNK)], idx_v)

        def group(g, _):
            rows = g * L + lanes
            acc = [jnp.ones((L,), jnp.float32) for _ in range(NACC)]
            for j in range(H):
                iv = plsc.load_gather(
                    idx_v, [rows, jnp.full((L,), j, jnp.int32)])
                v = plsc.load_gather(table_v, [iv])
                acc[j % NACC] = acc[j % NACC] * (1.0 + jnp.exp(v))
            p = (acc[0] * acc[1]) * (acc[2] * acc[3])
            no = 1.0 - 1.0 / p
            no = jnp.minimum(jnp.maximum(no, 0.0001), 0.99999)
            out_v[pl.ds(c * CHUNK + g * L, L)] = no
            return 0

        lax.fori_loop(0, GROUPS_PER_CHUNK, group, 0)

    pltpu.sync_copy(out_v, out_hbm.at[pl.ds(base, ROWS_PER_W)])


@jax.jit
def kernel(rules, relation, table):
    del relation
    tbl = table[:, 0].at[PAD_TOK].set(-jnp.inf)
    tbl = jnp.concatenate([tbl, jnp.zeros((TBL_PAD - (LEN_RULES + 1),), jnp.float32)])
    run = pl.kernel(
        _body,
        out_type=jax.ShapeDtypeStruct((B,), jnp.float32),
        mesh=plsc.VectorSubcoreMesh(
            core_axis_name="c", subcore_axis_name="s",
            num_cores=NC, num_subcores=NS,
        ),
        compiler_params=pltpu.CompilerParams(
            needs_layout_passes=False, use_tc_tiling_on_sc=True),
        scratch_types=[
            pltpu.VMEM((TBL_PAD,), jnp.float32),
            pltpu.VMEM((CHUNK, H), jnp.int32),
            pltpu.VMEM((ROWS_PER_W,), jnp.float32),
        ],
    )
    return run(rules, tbl).reshape(B, 1)


# speedup vs baseline: 1.1437x; 1.0818x over previous
import jax
import jax.numpy as jnp
from jax import lax
from jax.experimental import pallas as pl
from jax.experimental.pallas import tpu as pltpu
from jax.experimental.pallas import tpu_sc as plsc

B = 16384
H = 50
LEN_RULES = 100000
PAD_TOK = LEN_RULES
TBL_PAD = 100032
TBL_Q = TBL_PAD // 4
NC, NS, L = 2, 16, 16
NW = NC * NS
ROWS_PER_W = B // NW
CHUNK = 64
NCHUNK = ROWS_PER_W // CHUNK
GROUPS_PER_CHUNK = CHUNK // L
NACC = 4


def _body(rules_hbm, table_hbm, out_hbm, table_v, idx0, idx1, out_v,
          sem_t, sem0, sem1):
    wid = lax.axis_index("s") * NC + lax.axis_index("c")
    base = wid * ROWS_PER_W
    lanes = lax.iota(jnp.int32, L)

    idx_bufs = [idx0, idx1]
    sems = [sem0, sem1]

    def start(c):
        return pltpu.async_copy(
            rules_hbm.at[pl.ds(base + c * CHUNK, CHUNK), pl.ds(0, H)],
            idx_bufs[c % 2], sems[c % 2])

    with jax.named_scope("stage_start"):
        cp_t = [
            pltpu.async_copy(table_hbm.at[pl.ds(q * TBL_Q, TBL_Q)],
                             table_v.at[pl.ds(q * TBL_Q, TBL_Q)], sem_t)
            for q in range(4)
        ]
        cps = {0: start(0), 1: start(1)}
        for cp in cp_t:
            cp.wait()

    for c in range(NCHUNK):
        with jax.named_scope("chunk"):
            cps[c].wait()
            if c + 2 < NCHUNK:
                cps[c + 2] = start(c + 2)
            idx_v = idx_bufs[c % 2]

            @plsc.parallel_loop(0, GROUPS_PER_CHUNK, 1)
            def group(g):
                rows = g * L + lanes
                acc = [jnp.ones((L,), jnp.float32) for _ in range(NACC)]
                for j in range(H):
                    iv = plsc.load_gather(
                        idx_v, [rows, jnp.full((L,), j, jnp.int32)])
                    v = plsc.load_gather(table_v, [iv])
                    acc[j % NACC] = acc[j % NACC] * v
                p = (acc[0] * acc[1]) * (acc[2] * acc[3])
                no = 1.0 - 1.0 / p
                no = jnp.minimum(jnp.maximum(no, 0.0001), 0.99999)
                out_v[pl.ds(c * CHUNK + g * L, L)] = no

    with jax.named_scope("store_out"):
        pltpu.sync_copy(out_v, out_hbm.at[pl.ds(base, ROWS_PER_W)])


@jax.jit
def kernel(rules, relation, table):
    del relation
    tbl = 1.0 + jnp.exp(table[:, 0].at[PAD_TOK].set(-jnp.inf))
    tbl = jnp.concatenate([tbl, jnp.ones((TBL_PAD - (LEN_RULES + 1),), jnp.float32)])
    run = pl.kernel(
        _body,
        out_type=jax.ShapeDtypeStruct((B,), jnp.float32),
        mesh=plsc.VectorSubcoreMesh(
            core_axis_name="c", subcore_axis_name="s",
            num_cores=NC, num_subcores=NS,
        ),
        compiler_params=pltpu.CompilerParams(
            needs_layout_passes=False, use_tc_tiling_on_sc=True),
        scratch_types=[
            pltpu.VMEM((TBL_PAD,), jnp.float32),
            pltpu.VMEM((CHUNK, H), jnp.int32),
            pltpu.VMEM((CHUNK, H), jnp.int32),
            pltpu.VMEM((ROWS_PER_W,), jnp.float32),
            pltpu.SemaphoreType.DMA,
            pltpu.SemaphoreType.DMA,
            pltpu.SemaphoreType.DMA,
        ],
    )
    return run(rules, tbl).reshape(B, 1)
